# baseline (speedup 1.0000x reference)
import functools
import math

import jax
import jax.numpy as jnp
from jax import lax
from jax.experimental import pallas as pl
from jax.experimental.pallas import tpu as pltpu
from jax.experimental.pallas import tpu_sc as plsc

D_MODEL = 64
SCALE = math.sqrt(D_MODEL)
NUM_CORES = 2
NUM_SUBCORES = 16
NUM_WORKERS = NUM_CORES * NUM_SUBCORES
BBLK = 256
GATHER = 128
NG = BBLK // GATHER
VBLK = 2048


def _transpose_block(tin, tout):
    tout[...] = tin[...].T


def _tc_transpose(table_t):
    v = table_t.shape[1]
    return pl.pallas_call(
        _transpose_block,
        grid=(pl.cdiv(v, VBLK),),
        in_specs=[pl.BlockSpec((D_MODEL, VBLK), lambda i: (0, i))],
        out_specs=pl.BlockSpec((VBLK, D_MODEL), lambda i: (i, 0)),
        out_shape=jax.ShapeDtypeStruct((v, D_MODEL), jnp.float32),
    )(table_t)


@functools.partial(jax.jit, static_argnums=(2, 3))
def _embedding_lookup(xt, table_t, tsteps, batch):
    table_rm = _tc_transpose(table_t)
    n_pairs = tsteps * (batch // BBLK)
    p_per_w = n_pairs // NUM_WORKERS
    idx_rows = p_per_w * NG
    nbb = batch // BBLK
    idx3 = xt.reshape(tsteps * batch // GATHER, GATHER)
    mesh = plsc.VectorSubcoreMesh(core_axis_name="c", subcore_axis_name="s")

    @functools.partial(
        pl.kernel,
        mesh=mesh,
        out_type=jax.ShapeDtypeStruct((tsteps, D_MODEL, batch), jnp.float32),
        scratch_types=[
            pltpu.VMEM((idx_rows, GATHER), jnp.int32),
            [pltpu.VMEM((BBLK, D_MODEL), jnp.float32) for _ in range(2)],
            [pltpu.VMEM((D_MODEL, BBLK), jnp.float32) for _ in range(2)],
            [pltpu.SemaphoreType.DMA for _ in range(2)],
            [pltpu.SemaphoreType.DMA for _ in range(2)],
        ],
        compiler_params=pltpu.CompilerParams(
            needs_layout_passes=False, use_tc_tiling_on_sc=False
        ),
    )
    def k(idx_hbm, table_hbm, out_hbm, idx_v, rows, obufs, gsems, osems):
        wid = lax.axis_index("s") * NUM_CORES + lax.axis_index("c")
        base_p = wid * p_per_w
        pltpu.sync_copy(idx_hbm.at[pl.ds(wid * idx_rows, idx_rows)], idx_v)
        riota = lax.iota(jnp.int32, 16)

        def fire_gather(p, b):
            for j in range(NG):
                pltpu.async_copy(
                    table_hbm.at[idx_v.at[p * NG + j]],
                    rows[b].at[pl.ds(j * GATHER, GATHER)],
                    gsems[b],
                )

        def drain(sem, b):
            pltpu.make_async_copy(table_hbm.at[pl.ds(0, BBLK)], rows[b], sem).wait()

        def transpose_scale(b):
            rv, ob = rows[b], obufs[b]

            def body(d, carry):
                col = jnp.broadcast_to(d, (16,))
                for j in range(16):
                    vals = plsc.load_gather(rv, [riota + (16 * j), col])
                    ob[d, pl.ds(16 * j, 16)] = vals * SCALE
                return carry

            lax.fori_loop(0, D_MODEL, body, 0)

        fire_gather(0, 0)
        if p_per_w > 1:
            fire_gather(1, 1)

        def loop_body(c, carry):
            for b in range(2):
                p = c + b
                g = base_p + p
                t = g // nbb
                bb = g - t * nbb
                drain(gsems[b], b)
                transpose_scale(b)
                pltpu.async_copy(
                    obufs[b],
                    out_hbm.at[t, :, pl.ds(bb * BBLK, BBLK)],
                    osems[b],
                )
                pltpu.make_async_copy(
                    out_hbm.at[0, :, pl.ds(0, BBLK)], obufs[b], osems[b]
                ).wait()

                @pl.when(p + 2 < p_per_w)
                def _():
                    fire_gather(p + 2, b)

            return carry

        lax.fori_loop(0, p_per_w // 2, lambda i, c: loop_body(i * 2, c), 0)

    return k(idx3, table_rm)


def kernel(x, table):
    b0, b1 = x.shape
    out_p = _embedding_lookup(x.T.astype(jnp.int32), table.T, b1, b0)
    return jnp.transpose(out_p, (2, 0, 1))


# speedup vs baseline: 1.4563x; 1.4563x over previous
import functools
import math

import jax
import jax.numpy as jnp
from jax import lax
from jax.experimental import pallas as pl
from jax.experimental.pallas import tpu as pltpu
from jax.experimental.pallas import tpu_sc as plsc

D_MODEL = 64
SCALE = math.sqrt(D_MODEL)
NUM_CORES = 2
NUM_SUBCORES = 16
NUM_WORKERS = NUM_CORES * NUM_SUBCORES
CHUNK = 512
GATHER = 128
NG = CHUNK // GATHER
NBUF = 2
VBLK = 2048
OBLK = 2048


def _transpose_2d(tin, tout):
    tout[...] = tin[...].T


def _tc_transpose_table(table_t):
    v = table_t.shape[1]
    return pl.pallas_call(
        _transpose_2d,
        grid=(pl.cdiv(v, VBLK),),
        in_specs=[pl.BlockSpec((D_MODEL, VBLK), lambda i: (0, i))],
        out_specs=pl.BlockSpec((VBLK, D_MODEL), lambda i: (i, 0)),
        out_shape=jax.ShapeDtypeStruct((v, D_MODEL), jnp.float32),
    )(table_t)


def _transpose_out_block(tin, tout):
    tout[0] = tin[0].T


def _tc_transpose_out(flat3):
    tsteps, batch, _ = flat3.shape
    return pl.pallas_call(
        _transpose_out_block,
        grid=(tsteps, batch // OBLK),
        in_specs=[pl.BlockSpec((1, OBLK, D_MODEL), lambda t, j: (t, j, 0))],
        out_specs=pl.BlockSpec((1, D_MODEL, OBLK), lambda t, j: (t, 0, j)),
        out_shape=jax.ShapeDtypeStruct((tsteps, D_MODEL, batch), jnp.float32),
    )(flat3)


def _sc_gather_scale(idx3, table_rm, batch):
    b_per_w = batch // NUM_WORKERS
    n_chunks = b_per_w // CHUNK
    idx_rows = b_per_w // GATHER
    mesh = plsc.VectorSubcoreMesh(core_axis_name="c", subcore_axis_name="s")

    @functools.partial(
        pl.kernel,
        mesh=mesh,
        out_type=jax.ShapeDtypeStruct((batch, D_MODEL), jnp.float32),
        scratch_types=[
            pltpu.VMEM((idx_rows, GATHER), jnp.int32),
            [pltpu.VMEM((CHUNK, D_MODEL), jnp.float32) for _ in range(NBUF)],
            [pltpu.SemaphoreType.DMA for _ in range(NBUF)],
            [pltpu.SemaphoreType.DMA for _ in range(NBUF)],
        ],
        compiler_params=pltpu.CompilerParams(use_tc_tiling_on_sc=False),
    )
    def k(idx_hbm, table_hbm, out_hbm, idx_v, rows, gsems, osems):
        wid = lax.axis_index("s") * NUM_CORES + lax.axis_index("c")
        base_g = wid * idx_rows

        pltpu.sync_copy(idx_hbm.at[pl.ds(base_g, idx_rows)], idx_v)

        def fire_gather(cc, b):
            for j in range(NG):
                pltpu.async_copy(
                    table_hbm.at[idx_v.at[cc * NG + j]],
                    rows[b].at[pl.ds(j * GATHER, GATHER)],
                    gsems[b],
                )

        def wait_chunk(sem, b):
            pltpu.make_async_copy(table_hbm.at[pl.ds(0, CHUNK)], rows[b], sem).wait()

        def scale(b):
            def body(i, carry):
                for r in range(4):
                    for j in range(D_MODEL // 16):
                        sl = pl.ds(j * 16, 16)
                        rows[b][i * 4 + r, sl] = rows[b][i * 4 + r, sl] * SCALE
                return carry

            lax.fori_loop(0, CHUNK // 4, body, 0)

        for b in range(NBUF):
            fire_gather(b, b)

        def loop_body(c, carry):
            for b in range(NBUF):
                cc = c + b
                wait_chunk(gsems[b], b)
                scale(b)
                pltpu.async_copy(
                    rows[b],
                    out_hbm.at[pl.ds((base_g + cc * NG) * GATHER, CHUNK)],
                    osems[b],
                )
                wait_chunk(osems[b], b)

                @pl.when(cc + NBUF < n_chunks)
                def _():
                    fire_gather(cc + NBUF, b)

            return carry

        lax.fori_loop(0, n_chunks // NBUF, lambda i, c: loop_body(i * NBUF, c), 0)

    return k(idx3, table_rm)


@functools.partial(jax.jit, static_argnums=(2, 3))
def _embedding_lookup(xt, table_t, tsteps, batch):
    table_rm = _tc_transpose_table(table_t)
    idx3 = xt.reshape(tsteps * batch // GATHER, GATHER)
    flat = _sc_gather_scale(idx3, table_rm, tsteps * batch)
    return _tc_transpose_out(flat.reshape(tsteps, batch, D_MODEL))


def kernel(x, table):
    b0, b1 = x.shape
    out_p = _embedding_lookup(x.T.astype(jnp.int32), table.T, b1, b0)
    return jnp.transpose(out_p, (2, 0, 1))


# speedup vs baseline: 1.7964x; 1.2335x over previous
import functools
import math

import jax
import jax.numpy as jnp
from jax import lax
from jax.experimental import pallas as pl
from jax.experimental.pallas import tpu as pltpu
from jax.experimental.pallas import tpu_sc as plsc

D_MODEL = 64
SCALE = math.sqrt(D_MODEL)
NUM_CORES = 2
NUM_SUBCORES = 16
NUM_WORKERS = NUM_CORES * NUM_SUBCORES
CHUNK = 512
GATHER = 128
NG = CHUNK // GATHER
NBUF = 2
VBLK = 32768
OBLK = 4096


def _transpose_2d(tin, tout):
    tout[...] = tin[...].T


def _tc_transpose_table(table_t):
    v = table_t.shape[1]
    return pl.pallas_call(
        _transpose_2d,
        grid=(pl.cdiv(v, VBLK),),
        in_specs=[pl.BlockSpec((D_MODEL, VBLK), lambda i: (0, i))],
        out_specs=pl.BlockSpec((VBLK, D_MODEL), lambda i: (i, 0)),
        out_shape=jax.ShapeDtypeStruct((v, D_MODEL), jnp.float32),
    )(table_t)


def _transpose_out_block(tin, tout):
    tout[0] = tin[0].T


def _tc_transpose_out(flat3):
    tsteps, batch, _ = flat3.shape
    return pl.pallas_call(
        _transpose_out_block,
        grid=(tsteps, batch // OBLK),
        in_specs=[pl.BlockSpec((1, OBLK, D_MODEL), lambda t, j: (t, j, 0))],
        out_specs=pl.BlockSpec((1, D_MODEL, OBLK), lambda t, j: (t, 0, j)),
        out_shape=jax.ShapeDtypeStruct((tsteps, D_MODEL, batch), jnp.float32),
    )(flat3)


def _sc_gather_scale(idx3, table_rm, batch):
    b_per_w = batch // NUM_WORKERS
    n_chunks = b_per_w // CHUNK
    idx_rows = b_per_w // GATHER
    mesh = plsc.VectorSubcoreMesh(core_axis_name="c", subcore_axis_name="s")

    @functools.partial(
        pl.kernel,
        mesh=mesh,
        out_type=jax.ShapeDtypeStruct((batch, D_MODEL), jnp.float32),
        scratch_types=[
            pltpu.VMEM((idx_rows, GATHER), jnp.int32),
            [pltpu.VMEM((CHUNK, D_MODEL), jnp.float32) for _ in range(NBUF)],
            [pltpu.SemaphoreType.DMA for _ in range(NBUF)],
            [pltpu.SemaphoreType.DMA for _ in range(NBUF)],
        ],
        compiler_params=pltpu.CompilerParams(use_tc_tiling_on_sc=False),
    )
    def k(idx_hbm, table_hbm, out_hbm, idx_v, rows, gsems, osems):
        wid = lax.axis_index("s") * NUM_CORES + lax.axis_index("c")
        base_g = wid * idx_rows

        pltpu.sync_copy(idx_hbm.at[pl.ds(base_g, idx_rows)], idx_v)

        def fire_gather(cc, b):
            for j in range(NG):
                pltpu.async_copy(
                    table_hbm.at[idx_v.at[cc * NG + j]],
                    rows[b].at[pl.ds(j * GATHER, GATHER)],
                    gsems[b],
                )

        def wait_chunk(sem, b):
            pltpu.make_async_copy(table_hbm.at[pl.ds(0, CHUNK)], rows[b], sem).wait()

        def scale(b):
            def body(i, carry):
                for r in range(4):
                    for j in range(D_MODEL // 16):
                        sl = pl.ds(j * 16, 16)
                        rows[b][i * 4 + r, sl] = rows[b][i * 4 + r, sl] * SCALE
                return carry

            lax.fori_loop(0, CHUNK // 4, body, 0)

        for b in range(NBUF):
            fire_gather(b, b)

        def loop_body(c, carry):
            for b in range(NBUF):
                cc = c + b
                wait_chunk(gsems[b], b)
                scale(b)
                pltpu.async_copy(
                    rows[b],
                    out_hbm.at[pl.ds((base_g + cc * NG) * GATHER, CHUNK)],
                    osems[b],
                )
                wait_chunk(osems[b], b)

                @pl.when(cc + NBUF < n_chunks)
                def _():
                    fire_gather(cc + NBUF, b)

            return carry

        lax.fori_loop(0, n_chunks // NBUF, lambda i, c: loop_body(i * NBUF, c), 0)

    return k(idx3, table_rm)


@functools.partial(jax.jit, static_argnums=(2, 3))
def _embedding_lookup(xt, table_t, tsteps, batch):
    table_rm = _tc_transpose_table(table_t)
    idx3 = xt.reshape(tsteps * batch // GATHER, GATHER)
    flat = _sc_gather_scale(idx3, table_rm, tsteps * batch)
    return _tc_transpose_out(flat.reshape(tsteps, batch, D_MODEL))


def kernel(x, table):
    b0, b1 = x.shape
    out_p = _embedding_lookup(x.T.astype(jnp.int32), table.T, b1, b0)
    return jnp.transpose(out_p, (2, 0, 1))


# speedup vs baseline: 3.5714x; 1.9881x over previous
import functools
import math

import jax
import jax.numpy as jnp
from jax import lax
from jax.experimental import pallas as pl
from jax.experimental.pallas import tpu as pltpu
from jax.experimental.pallas import tpu_sc as plsc

D_MODEL = 64
SCALE = math.sqrt(D_MODEL)
NUM_CORES = 2
NUM_SUBCORES = 16
NUM_WORKERS = NUM_CORES * NUM_SUBCORES
CHUNK = 512
GATHER = 128
NG = CHUNK // GATHER
NBUF = 2
VBLK = 8192
OBLK = 4096


def _transpose_table_block(tin, tout):
    a = tin[...].T
    tout[...] = jnp.concatenate([a[: VBLK // 2], a[VBLK // 2 :]], axis=1)


def _tc_transpose_table(table_t):
    v = table_t.shape[1]
    nblk = pl.cdiv(v, VBLK)
    return pl.pallas_call(
        _transpose_table_block,
        grid=(nblk,),
        in_specs=[pl.BlockSpec((D_MODEL, VBLK), lambda i: (0, i))],
        out_specs=pl.BlockSpec((VBLK * D_MODEL // 128, 128), lambda i: (i, 0)),
        out_shape=jax.ShapeDtypeStruct(
            (nblk * VBLK * D_MODEL // 128, 128), jnp.float32
        ),
    )(table_t)


def _transpose_out_block(tin, tout):
    tout[0] = jnp.concatenate(
        [tin[:, :D_MODEL].T, tin[:, D_MODEL:].T], axis=1
    )


def _tc_transpose_out(flat2, tsteps, batch):
    return pl.pallas_call(
        _transpose_out_block,
        grid=(tsteps,),
        in_specs=[pl.BlockSpec((OBLK * D_MODEL // 128, 128), lambda i: (i, 0))],
        out_specs=pl.BlockSpec((1, D_MODEL, OBLK), lambda i: (i, 0, 0)),
        out_shape=jax.ShapeDtypeStruct((tsteps, D_MODEL, batch), jnp.float32),
    )(flat2)


def _sc_gather_scale(idx3, table_rm, batch):
    b_per_w = batch // NUM_WORKERS
    n_chunks = b_per_w // CHUNK
    idx_rows = b_per_w // GATHER
    mesh = plsc.VectorSubcoreMesh(core_axis_name="c", subcore_axis_name="s")

    @functools.partial(
        pl.kernel,
        mesh=mesh,
        out_type=jax.ShapeDtypeStruct((batch * D_MODEL // 128, 128), jnp.float32),
        scratch_types=[
            pltpu.VMEM((idx_rows, GATHER), jnp.int32),
            [pltpu.VMEM((CHUNK, D_MODEL), jnp.float32) for _ in range(NBUF)],
            [pltpu.SemaphoreType.DMA for _ in range(NBUF)],
            [pltpu.SemaphoreType.DMA for _ in range(NBUF)],
        ],
        compiler_params=pltpu.CompilerParams(use_tc_tiling_on_sc=False),
    )
    def k(idx_hbm, table_hbm, out_hbm, idx_v, rows, gsems, osems):
        wid = lax.axis_index("s") * NUM_CORES + lax.axis_index("c")
        base_g = wid * idx_rows

        pltpu.sync_copy(idx_hbm.at[pl.ds(base_g, idx_rows)], idx_v)

        def xform(i, carry):
            for j in range(GATHER // 16):
                sl = pl.ds(j * 16, 16)
                v = idx_v[i, sl]
                o = v & (VBLK - 1)
                idx_v[i, sl] = (
                    v - o + ((o & (VBLK // 2 - 1)) << 1) + (o >> (VBLK // 2).bit_length() - 1)
                )
            return carry

        lax.fori_loop(0, idx_rows, xform, 0)

        def fire_gather(cc, b):
            for j in range(NG):
                pltpu.async_copy(
                    table_hbm.at[idx_v.at[cc * NG + j]],
                    rows[b].at[pl.ds(j * GATHER, GATHER)],
                    gsems[b],
                )

        def wait_chunk(sem, b):
            pltpu.make_async_copy(table_hbm.at[pl.ds(0, CHUNK)], rows[b], sem).wait()

        def scale(b):
            def body(i, carry):
                for r in range(4):
                    for j in range(D_MODEL // 16):
                        sl = pl.ds(j * 16, 16)
                        rows[b][i * 4 + r, sl] = rows[b][i * 4 + r, sl] * SCALE
                return carry

            lax.fori_loop(0, CHUNK // 4, body, 0)

        for b in range(NBUF):
            fire_gather(b, b)

        def loop_body(c, carry):
            for b in range(NBUF):
                cc = c + b
                wait_chunk(gsems[b], b)
                scale(b)
                r0 = (base_g + cc * NG) * GATHER
                o0 = lax.rem(r0, OBLK)
                h = o0 // (OBLK // 2)
                wk0 = (r0 // OBLK) * (OBLK // 2) + lax.rem(o0, OBLK // 2)
                pltpu.async_copy(
                    rows[b],
                    out_hbm.at[pl.ds(wk0, CHUNK), pl.ds(h * D_MODEL, D_MODEL)],
                    osems[b],
                )
                wait_chunk(osems[b], b)

                @pl.when(cc + NBUF < n_chunks)
                def _():
                    fire_gather(cc + NBUF, b)

            return carry

        lax.fori_loop(0, n_chunks // NBUF, lambda i, c: loop_body(i * NBUF, c), 0)

    return k(idx3, table_rm)


@functools.partial(jax.jit, static_argnums=(2, 3))
def _embedding_lookup(xt, table_t, tsteps, batch):
    wide = _tc_transpose_table(table_t)
    table_rm = wide.reshape(wide.shape[0] * 2, D_MODEL)
    idx3 = xt.reshape(tsteps * batch // GATHER, GATHER)
    flat2 = _sc_gather_scale(idx3, table_rm, tsteps * batch)
    return _tc_transpose_out(flat2, tsteps, batch)


def kernel(x, table):
    b0, b1 = x.shape
    out_p = _embedding_lookup(x.T.astype(jnp.int32), table.T, b1, b0)
    return jnp.transpose(out_p, (2, 0, 1))


# speedup vs baseline: 4.3050x; 1.2054x over previous
import functools
import math

import jax
import jax.numpy as jnp
from jax import lax
from jax.experimental import pallas as pl
from jax.experimental.pallas import tpu as pltpu
from jax.experimental.pallas import tpu_sc as plsc

D_MODEL = 64
SCALE = math.sqrt(D_MODEL)
NUM_CORES = 2
NUM_SUBCORES = 16
NUM_WORKERS = NUM_CORES * NUM_SUBCORES
CHUNK = 512
GATHER = 128
NG = CHUNK // GATHER
NBUF = 2
VBLK = 16384
OBLK = 4096


def _transpose_table_block(tin, tout):
    a = tin[...].T
    tout[...] = jnp.concatenate([a[: VBLK // 2], a[VBLK // 2 :]], axis=1)


def _tc_transpose_table(table_t):
    v = table_t.shape[1]
    nblk = pl.cdiv(v, VBLK)
    return pl.pallas_call(
        _transpose_table_block,
        grid=(nblk,),
        in_specs=[pl.BlockSpec((D_MODEL, VBLK), lambda i: (0, i))],
        out_specs=pl.BlockSpec((VBLK * D_MODEL // 128, 128), lambda i: (i, 0)),
        out_shape=jax.ShapeDtypeStruct(
            (nblk * VBLK * D_MODEL // 128, 128), jnp.float32
        ),
    )(table_t)


TPB = 4


def _transpose_out_block(tin, tout):
    half = OBLK * D_MODEL // 128
    for t in range(TPB):
        blk = tin[pl.ds(t * half, half), :]
        tout[t] = jnp.concatenate(
            [blk[:, :D_MODEL].T, blk[:, D_MODEL:].T], axis=1
        )


def _tc_transpose_out(flat2, tsteps, batch):
    return pl.pallas_call(
        _transpose_out_block,
        grid=(tsteps // TPB,),
        in_specs=[
            pl.BlockSpec((TPB * OBLK * D_MODEL // 128, 128), lambda i: (i, 0))
        ],
        out_specs=pl.BlockSpec((TPB, D_MODEL, OBLK), lambda i: (i, 0, 0)),
        out_shape=jax.ShapeDtypeStruct((tsteps, D_MODEL, batch), jnp.float32),
    )(flat2)


def _sc_gather_scale(idx3, table_rm, batch):
    b_per_w = batch // NUM_WORKERS
    n_chunks = b_per_w // CHUNK
    idx_rows = b_per_w // GATHER
    mesh = plsc.VectorSubcoreMesh(core_axis_name="c", subcore_axis_name="s")

    @functools.partial(
        pl.kernel,
        mesh=mesh,
        out_type=jax.ShapeDtypeStruct((batch * D_MODEL // 128, 128), jnp.float32),
        scratch_types=[
            pltpu.VMEM((idx_rows, GATHER), jnp.int32),
            [pltpu.VMEM((CHUNK, D_MODEL), jnp.float32) for _ in range(NBUF)],
            [pltpu.SemaphoreType.DMA for _ in range(NBUF)],
            [pltpu.SemaphoreType.DMA for _ in range(NBUF)],
        ],
        compiler_params=pltpu.CompilerParams(use_tc_tiling_on_sc=False),
    )
    def k(idx_hbm, table_hbm, out_hbm, idx_v, rows, gsems, osems):
        wid = lax.axis_index("s") * NUM_CORES + lax.axis_index("c")
        base_g = wid * idx_rows

        pltpu.sync_copy(idx_hbm.at[pl.ds(base_g, idx_rows)], idx_v)

        def xform(i, carry):
            for j in range(GATHER // 16):
                sl = pl.ds(j * 16, 16)
                v = idx_v[i, sl]
                o = v & (VBLK - 1)
                idx_v[i, sl] = (
                    v - o + ((o & (VBLK // 2 - 1)) << 1) + (o >> (VBLK // 2).bit_length() - 1)
                )
            return carry

        lax.fori_loop(0, idx_rows, xform, 0)

        def fire_gather(cc, b):
            for j in range(NG):
                pltpu.async_copy(
                    table_hbm.at[idx_v.at[cc * NG + j]],
                    rows[b].at[pl.ds(j * GATHER, GATHER)],
                    gsems[b],
                )

        def wait_chunk(sem, b):
            pltpu.make_async_copy(table_hbm.at[pl.ds(0, CHUNK)], rows[b], sem).wait()

        def scale(b):
            def body(i, carry):
                for r in range(4):
                    for j in range(D_MODEL // 16):
                        sl = pl.ds(j * 16, 16)
                        rows[b][i * 4 + r, sl] = rows[b][i * 4 + r, sl] * SCALE
                return carry

            lax.fori_loop(0, CHUNK // 4, body, 0)

        for b in range(NBUF):
            fire_gather(b, b)

        def loop_body(c, carry):
            for b in range(NBUF):
                cc = c + b
                wait_chunk(gsems[b], b)
                scale(b)
                r0 = (base_g + cc * NG) * GATHER
                o0 = lax.rem(r0, OBLK)
                h = o0 // (OBLK // 2)
                wk0 = (r0 // OBLK) * (OBLK // 2) + lax.rem(o0, OBLK // 2)
                pltpu.async_copy(
                    rows[b],
                    out_hbm.at[pl.ds(wk0, CHUNK), pl.ds(h * D_MODEL, D_MODEL)],
                    osems[b],
                )
                wait_chunk(osems[b], b)

                @pl.when(cc + NBUF < n_chunks)
                def _():
                    fire_gather(cc + NBUF, b)

            return carry

        lax.fori_loop(0, n_chunks // NBUF, lambda i, c: loop_body(i * NBUF, c), 0)

    return k(idx3, table_rm)


@functools.partial(jax.jit, static_argnums=(2, 3))
def _embedding_lookup(xt, table_t, tsteps, batch):
    wide = _tc_transpose_table(table_t)
    table_rm = wide.reshape(wide.shape[0] * 2, D_MODEL)
    idx3 = xt.reshape(tsteps * batch // GATHER, GATHER)
    flat2 = _sc_gather_scale(idx3, table_rm, tsteps * batch)
    return _tc_transpose_out(flat2, tsteps, batch)


def kernel(x, table):
    b0, b1 = x.shape
    out_p = _embedding_lookup(x.T.astype(jnp.int32), table.T, b1, b0)
    return jnp.transpose(out_p, (2, 0, 1))


# speedup vs baseline: 4.4906x; 1.0431x over previous
import functools
import math

import jax
import jax.numpy as jnp
from jax import lax
from jax.experimental import pallas as pl
from jax.experimental.pallas import tpu as pltpu
from jax.experimental.pallas import tpu_sc as plsc

D_MODEL = 64
SCALE = math.sqrt(D_MODEL)
NUM_CORES = 2
NUM_SUBCORES = 16
NUM_WORKERS = NUM_CORES * NUM_SUBCORES
CHUNK = 512
GATHER = 128
NG = CHUNK // GATHER
NBUF = 2
VBLK = 32768
OBLK = 4096


def _transpose_table_block(tin, tout):
    a = tin[...].T
    tout[...] = jnp.concatenate([a[: VBLK // 2], a[VBLK // 2 :]], axis=1)


def _tc_transpose_table(table_t):
    v = table_t.shape[1]
    nblk = pl.cdiv(v, VBLK)
    return pl.pallas_call(
        _transpose_table_block,
        grid=(nblk,),
        in_specs=[pl.BlockSpec((D_MODEL, VBLK), lambda i: (0, i))],
        out_specs=pl.BlockSpec((VBLK * D_MODEL // 128, 128), lambda i: (i, 0)),
        out_shape=jax.ShapeDtypeStruct(
            (nblk * VBLK * D_MODEL // 128, 128), jnp.float32
        ),
    )(table_t)


TPB = 8


def _transpose_out_block(tin, tout):
    half = OBLK * D_MODEL // 128
    for t in range(TPB):
        blk = tin[pl.ds(t * half, half), :]
        tout[t] = jnp.concatenate(
            [blk[:, :D_MODEL].T, blk[:, D_MODEL:].T], axis=1
        )


def _tc_transpose_out(flat2, tsteps, batch):
    return pl.pallas_call(
        _transpose_out_block,
        grid=(tsteps // TPB,),
        in_specs=[
            pl.BlockSpec((TPB * OBLK * D_MODEL // 128, 128), lambda i: (i, 0))
        ],
        out_specs=pl.BlockSpec((TPB, D_MODEL, OBLK), lambda i: (i, 0, 0)),
        out_shape=jax.ShapeDtypeStruct((tsteps, D_MODEL, batch), jnp.float32),
    )(flat2)


def _sc_gather_scale(idx3, table_rm, batch):
    b_per_w = batch // NUM_WORKERS
    n_chunks = b_per_w // CHUNK
    idx_rows = b_per_w // GATHER
    mesh = plsc.VectorSubcoreMesh(core_axis_name="c", subcore_axis_name="s")

    @functools.partial(
        pl.kernel,
        mesh=mesh,
        out_type=jax.ShapeDtypeStruct((batch * D_MODEL // 128, 128), jnp.float32),
        scratch_types=[
            pltpu.VMEM((idx_rows, GATHER), jnp.int32),
            [pltpu.VMEM((CHUNK, D_MODEL), jnp.float32) for _ in range(NBUF)],
            [pltpu.SemaphoreType.DMA for _ in range(NBUF)],
            [pltpu.SemaphoreType.DMA for _ in range(NBUF)],
        ],
        compiler_params=pltpu.CompilerParams(use_tc_tiling_on_sc=False),
    )
    def k(idx_hbm, table_hbm, out_hbm, idx_v, rows, gsems, osems):
        wid = lax.axis_index("s") * NUM_CORES + lax.axis_index("c")
        base_g = wid * idx_rows

        pltpu.sync_copy(idx_hbm.at[pl.ds(base_g, idx_rows)], idx_v)

        def xform(i, carry):
            for j in range(GATHER // 16):
                sl = pl.ds(j * 16, 16)
                v = idx_v[i, sl]
                o = v & (VBLK - 1)
                idx_v[i, sl] = (
                    v - o + ((o & (VBLK // 2 - 1)) << 1) + (o >> (VBLK // 2).bit_length() - 1)
                )
            return carry

        lax.fori_loop(0, idx_rows, xform, 0)

        def fire_gather(cc, b):
            for j in range(NG):
                pltpu.async_copy(
                    table_hbm.at[idx_v.at[cc * NG + j]],
                    rows[b].at[pl.ds(j * GATHER, GATHER)],
                    gsems[b],
                )

        def wait_chunk(sem, b):
            pltpu.make_async_copy(table_hbm.at[pl.ds(0, CHUNK)], rows[b], sem).wait()

        def scale(b):
            def body(i, carry):
                for r in range(4):
                    for j in range(D_MODEL // 16):
                        sl = pl.ds(j * 16, 16)
                        rows[b][i * 4 + r, sl] = rows[b][i * 4 + r, sl] * SCALE
                return carry

            lax.fori_loop(0, CHUNK // 4, body, 0)

        for b in range(NBUF):
            fire_gather(b, b)

        def loop_body(c, carry):
            for b in range(NBUF):
                cc = c + b
                wait_chunk(gsems[b], b)
                scale(b)
                r0 = (base_g + cc * NG) * GATHER
                o0 = lax.rem(r0, OBLK)
                h = o0 // (OBLK // 2)
                wk0 = (r0 // OBLK) * (OBLK // 2) + lax.rem(o0, OBLK // 2)
                pltpu.async_copy(
                    rows[b],
                    out_hbm.at[pl.ds(wk0, CHUNK), pl.ds(h * D_MODEL, D_MODEL)],
                    osems[b],
                )
                wait_chunk(osems[b], b)

                @pl.when(cc + NBUF < n_chunks)
                def _():
                    fire_gather(cc + NBUF, b)

            return carry

        lax.fori_loop(0, n_chunks // NBUF, lambda i, c: loop_body(i * NBUF, c), 0)

    return k(idx3, table_rm)


@functools.partial(jax.jit, static_argnums=(2, 3))
def _embedding_lookup(xt, table_t, tsteps, batch):
    wide = _tc_transpose_table(table_t)
    table_rm = wide.reshape(wide.shape[0] * 2, D_MODEL)
    idx3 = xt.reshape(tsteps * batch // GATHER, GATHER)
    flat2 = _sc_gather_scale(idx3, table_rm, tsteps * batch)
    return _tc_transpose_out(flat2, tsteps, batch)


def kernel(x, table):
    b0, b1 = x.shape
    out_p = _embedding_lookup(x.T.astype(jnp.int32), table.T, b1, b0)
    return jnp.transpose(out_p, (2, 0, 1))
